# Initial kernel scaffold; baseline (speedup 1.0000x reference)
import jax
import jax.numpy as jnp
from jax.experimental import pallas as pl


def kernel(index, values):
    raise NotImplementedError("write your pallas kernel here")


# speedup vs baseline: 3.9611x; 3.9611x over previous
import functools

import jax
import jax.numpy as jnp
from jax import lax
from jax.experimental import pallas as pl
from jax.experimental.pallas import tpu as pltpu
from jax.experimental.pallas import tpu_sc as plsc

N_COLS = 16384
NNZ = 268435
D = 64

NC = 2
NS = 16
NW = NC * NS

CHUNK = 400
PER_T = 16800
NNZ_PAD = PER_T * NS
NCHUNK_A = PER_T // CHUNK
ATAIL_C = NCHUNK_A - 1
PER_W = NNZ_PAD // NW
NCHUNK_G = PER_W // CHUNK
GTAIL_W = NW - 1
GTAIL_C = NCHUNK_G - 1
TAIL_N = NNZ - (NNZ_PAD - CHUNK)

ST = 16640
ROWS_PT = ST // NS
REM = ROWS_PT - 2 * CHUNK
W = D + 16

_mesh = plsc.VectorSubcoreMesh(core_axis_name="c", subcore_axis_name="s")
_f32 = jnp.float32


@functools.partial(
    pl.kernel,
    out_type=jax.ShapeDtypeStruct((NNZ, D), _f32),
    mesh=_mesh,
    compiler_params=pltpu.CompilerParams(use_tc_tiling_on_sc=False),
    scratch_types=[
        pltpu.VMEM_SHARED((ST, W), _f32),
        pltpu.VMEM((CHUNK,), jnp.int32),
        pltpu.VMEM((CHUNK, W), _f32),
        pltpu.SemaphoreType.DMA,
    ],
)
def _pool(seg_hbm, val_hbm, zrow_hbm, ones_hbm, out_hbm,
          tab_sh, idx_v, buf_v, sem):
    core = lax.axis_index("c")
    sub = lax.axis_index("s")

    pltpu.sync_copy(zrow_hbm, buf_v)
    row0 = sub * ROWS_PT
    pltpu.sync_copy(buf_v, tab_sh.at[pl.ds(row0, CHUNK)])
    pltpu.sync_copy(buf_v, tab_sh.at[pl.ds(row0 + CHUNK, CHUNK)])
    pltpu.sync_copy(buf_v.at[pl.ds(0, REM)],
                    tab_sh.at[pl.ds(row0 + 2 * CHUNK, REM)])
    pltpu.sync_copy(ones_hbm, buf_v.at[:, pl.ds(D, 16)])
    plsc.subcore_barrier()

    def abody(ci, _):
        base = sub * PER_T + ci * CHUNK
        pltpu.sync_copy(seg_hbm.at[pl.ds(base, CHUNK)], idx_v)
        is_tail = jnp.logical_and(sub == NS - 1, ci == ATAIL_C)

        @pl.when(jnp.logical_not(is_tail))
        def _():
            pltpu.sync_copy(val_hbm.at[pl.ds(base, CHUNK)],
                            buf_v.at[:, pl.ds(0, D)])

        @pl.when(is_tail)
        def _():
            pltpu.sync_copy(val_hbm.at[pl.ds(NNZ_PAD - CHUNK, TAIL_N)],
                            buf_v.at[pl.ds(0, TAIL_N), pl.ds(0, D)])

        pltpu.sync_copy(buf_v, tab_sh.at[idx_v], add=True)
        return ()

    lax.fori_loop(0, NCHUNK_A, abody, ())
    plsc.subcore_barrier()

    def cblock(brow, nrows):
        pltpu.sync_copy(tab_sh.at[pl.ds(brow, nrows)], buf_v.at[pl.ds(0, nrows)])

        def cbody(r, _):
            inv = 1.0 / buf_v[r, pl.ds(D, 16)]
            for j in range(D // 16):
                buf_v[r, pl.ds(j * 16, 16)] = buf_v[r, pl.ds(j * 16, 16)] * inv
            return ()

        lax.fori_loop(0, nrows, cbody, ())
        pltpu.sync_copy(buf_v.at[pl.ds(0, nrows)], tab_sh.at[pl.ds(brow, nrows)])

    cblock(row0, CHUNK)
    cblock(row0 + CHUNK, CHUNK)
    cblock(row0 + 2 * CHUNK, REM)
    plsc.subcore_barrier()

    wid = sub * NC + core

    def gbody(ci, _):
        base = wid * PER_W + ci * CHUNK
        pltpu.sync_copy(seg_hbm.at[pl.ds(base, CHUNK)], idx_v)
        pltpu.async_copy(tab_sh.at[idx_v], buf_v, sem).wait()
        is_tail = jnp.logical_and(wid == GTAIL_W, ci == GTAIL_C)

        @pl.when(jnp.logical_not(is_tail))
        def _():
            pltpu.sync_copy(buf_v.at[:, pl.ds(0, D)],
                            out_hbm.at[pl.ds(base, CHUNK)])

        @pl.when(is_tail)
        def _():
            pltpu.sync_copy(buf_v.at[pl.ds(0, TAIL_N), pl.ds(0, D)],
                            out_hbm.at[pl.ds(NNZ_PAD - CHUNK, TAIL_N)])
        return ()

    lax.fori_loop(0, NCHUNK_G, gbody, ())


def kernel(index, values):
    seg = index[1].astype(jnp.int32)
    seg_pad = jnp.concatenate(
        [seg, jnp.full((NNZ_PAD - NNZ,), N_COLS, dtype=jnp.int32)])
    zrow = jnp.zeros((CHUNK, W), dtype=_f32)
    ones = jnp.ones((CHUNK, 16), dtype=_f32)
    return _pool(seg_pad, values, zrow, ones)


# speedup vs baseline: 4.3929x; 1.1090x over previous
import functools

import jax
import jax.numpy as jnp
from jax import lax
from jax.experimental import pallas as pl
from jax.experimental.pallas import tpu as pltpu
from jax.experimental.pallas import tpu_sc as plsc

N_ROWS = 16384
N_COLS = 16384
NNZ = 268435
D = 64

NC = 2
NS = 16
NW = NC * NS

PER_W = 8400
NNZ_PAD = PER_W * NW
CHUNK = 400
NCHUNK = PER_W // CHUNK
TAIL_W = NW - 1
TAIL_C = (NNZ - TAIL_W * PER_W) // CHUNK
TAIL_N = NNZ - (TAIL_W * PER_W + TAIL_C * CHUNK)

ST = 16640
ROWS_PT = ST // NS
HALF = ROWS_PT // 2
REM = ROWS_PT - 2 * CHUNK
CW = 16

_mesh = plsc.VectorSubcoreMesh(core_axis_name="c", subcore_axis_name="s")
_f32 = jnp.float32


CH = 264
SLOT = 272
LOAD = 267
NCH_TOT = 1024
CPW = NCH_TOT // NW
NPAIR = CPW // 2
A_BASE = 268168
SPEC_K = A_BASE // CH + 1
SPEC_J0 = SPEC_K * CH - A_BASE
ZOFF = (0, SLOT, 2 * SLOT, ROWS_PT - SLOT)


@functools.partial(
    pl.kernel,
    out_type=(
        jax.ShapeDtypeStruct((NC, ST, D), _f32),
        jax.ShapeDtypeStruct((NC, ST, CW), _f32),
    ),
    mesh=_mesh,
    compiler_params=pltpu.CompilerParams(use_tc_tiling_on_sc=False),
    scratch_types=[
        pltpu.VMEM_SHARED((ST, D), _f32),
        pltpu.VMEM_SHARED((ST, CW), _f32),
        pltpu.VMEM((SLOT,), jnp.int32),
        pltpu.VMEM((SLOT,), jnp.int32),
        pltpu.VMEM((SLOT, D), _f32),
        pltpu.VMEM((SLOT, D), _f32),
        pltpu.VMEM((SLOT, CW), _f32),
        pltpu.SemaphoreType.DMA,
        pltpu.SemaphoreType.DMA,
        pltpu.SemaphoreType.DMA,
        pltpu.SemaphoreType.DMA,
    ],
)
def _accumulate(sched_hbm, val_hbm, zrow_hbm, zcnt_hbm, ones_hbm,
                psum_hbm, pcnt_hbm,
                acc_sh, cnt_sh, idx_a, idx_b, val_a, val_b, ones_v,
                sia, sva, sib, svb):
    core = lax.axis_index("c")
    sub = lax.axis_index("s")
    wid = sub * NC + core

    pltpu.sync_copy(zrow_hbm, val_a)
    pltpu.sync_copy(zcnt_hbm, ones_v)
    row0 = sub * ROWS_PT
    for z in ZOFF:
        pltpu.sync_copy(val_a, acc_sh.at[pl.ds(row0 + z, SLOT)])
        pltpu.sync_copy(ones_v, cnt_sh.at[pl.ds(row0 + z, SLOT)])
    pltpu.sync_copy(ones_hbm, ones_v)
    plsc.subcore_barrier()

    c0 = wid * CPW

    def start(c, idx_v, val_v, si, sv):
        base = jnp.minimum(c * CH, A_BASE)
        pltpu.async_copy(sched_hbm.at[pl.ds(c * SLOT, SLOT)], idx_v, si)
        pltpu.async_copy(val_hbm.at[pl.ds(base, LOAD)],
                         val_v.at[pl.ds(0, LOAD)], sv)

    def finish(c, idx_v, val_v, si, sv):
        base = jnp.minimum(c * CH, A_BASE)
        pltpu.make_async_copy(sched_hbm.at[pl.ds(c * SLOT, SLOT)],
                              idx_v, si).wait()
        pltpu.make_async_copy(val_hbm.at[pl.ds(base, LOAD)],
                              val_v.at[pl.ds(0, LOAD)], sv).wait()
        pltpu.sync_copy(val_v, acc_sh.at[idx_v], add=True)
        pltpu.sync_copy(ones_v, cnt_sh.at[idx_v], add=True)

    start(c0, idx_a, val_a, sia, sva)
    start(c0 + 1, idx_b, val_b, sib, svb)

    def body(t, _):
        ca = c0 + 2 * t
        finish(ca, idx_a, val_a, sia, sva)

        @pl.when(t < NPAIR - 1)
        def _():
            start(ca + 2, idx_a, val_a, sia, sva)

        finish(ca + 1, idx_b, val_b, sib, svb)

        @pl.when(t < NPAIR - 1)
        def _():
            start(ca + 3, idx_b, val_b, sib, svb)
        return ()

    lax.fori_loop(0, NPAIR, body, ())
    plsc.subcore_barrier()

    pltpu.sync_copy(acc_sh.at[pl.ds(row0, ROWS_PT)],
                    psum_hbm.at[core, pl.ds(row0, ROWS_PT)])
    pltpu.sync_copy(cnt_sh.at[pl.ds(row0, ROWS_PT)],
                    pcnt_hbm.at[core, pl.ds(row0, ROWS_PT)])


@functools.partial(
    pl.kernel,
    out_type=jax.ShapeDtypeStruct((ST, D), _f32),
    mesh=_mesh,
    compiler_params=pltpu.CompilerParams(use_tc_tiling_on_sc=False),
    scratch_types=[
        pltpu.VMEM((HALF, D), _f32),
        pltpu.VMEM((HALF, D), _f32),
        pltpu.VMEM((HALF, CW), _f32),
        pltpu.VMEM((HALF, CW), _f32),
    ],
)
def _combine(psum_hbm, pcnt_hbm, pooled_hbm, a_v, b_v, ca_v, cb_v):
    core = lax.axis_index("c")
    sub = lax.axis_index("s")
    wid = sub * NC + core
    row0 = wid * HALF
    pltpu.sync_copy(psum_hbm.at[0, pl.ds(row0, HALF)], a_v)
    pltpu.sync_copy(psum_hbm.at[1, pl.ds(row0, HALF)], b_v)
    pltpu.sync_copy(pcnt_hbm.at[0, pl.ds(row0, HALF)], ca_v)
    pltpu.sync_copy(pcnt_hbm.at[1, pl.ds(row0, HALF)], cb_v)

    def body(r, _):
        cvec = ca_v[r, pl.ds(0, CW)] + cb_v[r, pl.ds(0, CW)]
        invv = 1.0 / cvec
        for j in range(D // 16):
            s = a_v[r, pl.ds(j * 16, 16)] + b_v[r, pl.ds(j * 16, 16)]
            a_v[r, pl.ds(j * 16, 16)] = s * invv
        return ()

    lax.fori_loop(0, HALF, body, ())
    pltpu.sync_copy(a_v, pooled_hbm.at[pl.ds(row0, HALF)])


@functools.partial(
    pl.kernel,
    out_type=jax.ShapeDtypeStruct((NNZ, D), _f32),
    mesh=_mesh,
    compiler_params=pltpu.CompilerParams(use_tc_tiling_on_sc=False),
    scratch_types=[
        pltpu.VMEM((CHUNK,), jnp.int32),
        pltpu.VMEM((CHUNK, D), _f32),
        pltpu.SemaphoreType.DMA,
    ],
)
def _gather(seg_hbm, pooled_hbm, out_hbm, idx_v, row_v, sem):
    core = lax.axis_index("c")
    sub = lax.axis_index("s")
    wid = sub * NC + core

    def body(ci, _):
        base = wid * PER_W + ci * CHUNK
        pltpu.sync_copy(seg_hbm.at[pl.ds(base, CHUNK)], idx_v)
        pltpu.async_copy(pooled_hbm.at[idx_v], row_v, sem).wait()
        is_tail = jnp.logical_and(wid == TAIL_W, ci == TAIL_C)

        @pl.when(jnp.logical_not(is_tail))
        def _():
            pltpu.sync_copy(row_v, out_hbm.at[pl.ds(base, CHUNK)])

        @pl.when(is_tail)
        def _():
            pltpu.sync_copy(row_v.at[pl.ds(0, TAIL_N)],
                            out_hbm.at[pl.ds(TAIL_W * PER_W + TAIL_C * CHUNK,
                                             TAIL_N)])
        return ()

    lax.fori_loop(0, NCHUNK, body, ())


def kernel(index, values):
    seg = index[1].astype(jnp.int32)
    seg_pad = jnp.concatenate(
        [seg, jnp.full((NNZ_PAD - NNZ,), N_COLS, dtype=jnp.int32)])
    k = jnp.arange(NCH_TOT, dtype=jnp.int32)[:, None]
    j = jnp.arange(SLOT, dtype=jnp.int32)[None, :]
    base = jnp.minimum(k * CH, A_BASE)
    row = base + j
    valid = ((k < SPEC_K) & (j < CH)) | (
        (k == SPEC_K) & (j >= SPEC_J0) & (j < LOAD))
    sched = jnp.where(valid, seg[jnp.clip(row, 0, NNZ - 1)],
                      N_COLS).reshape(-1)
    zrow = jnp.zeros((SLOT, D), dtype=_f32)
    zcnt = jnp.zeros((SLOT, CW), dtype=_f32)
    ones = jnp.ones((SLOT, CW), dtype=_f32)
    psum, pcnt = _accumulate(sched, values, zrow, zcnt, ones)
    pooled = _combine(psum, pcnt)
    return _gather(seg_pad, pooled)
